# baseline (speedup 1.0000x reference)
import functools
import jax
import jax.numpy as jnp
from jax import lax
from jax.experimental import pallas as pl
from jax.experimental.pallas import tpu as pltpu
from jax.experimental.pallas import tpu_sc as plsc

_D = 64
_SCALE = 8.0

_info = plsc.get_sparse_core_info()
_NC, _NS = _info.num_cores, _info.num_subcores
_NW = _NC * _NS


def _make_kernel(V, BB, HQ):
    H = 8 * HQ
    mesh = plsc.VectorSubcoreMesh(core_axis_name="c", subcore_axis_name="s")

    @functools.partial(
        pl.kernel,
        mesh=mesh,
        out_type=jax.ShapeDtypeStruct((H, _D // 8, BB, 8, 128), jnp.float32),
        scratch_types=[
            pltpu.VMEM((HQ, 8, 128), jnp.int32),
            pltpu.VMEM((128, _D), jnp.float32),
            pltpu.VMEM((128, _D), jnp.float32),
            pltpu.VMEM((_D, 128), jnp.float32),
            pltpu.VMEM((_D, 128), jnp.float32),
            pltpu.SemaphoreType.DMA,
            pltpu.SemaphoreType.DMA,
            pltpu.SemaphoreType.DMA,
            pltpu.SemaphoreType.DMA,
        ],
        compiler_params=pltpu.CompilerParams(
            use_tc_tiling_on_sc=False, needs_layout_passes=False
        ),
    )
    def emb_kernel(
        table_hbm, idx_hbm, out_hbm,
        idx_v, rows0, rows1, ob0, ob1, sg0, sg1, sw0, sw1,
    ):
        w = lax.axis_index("s") * _NC + lax.axis_index("c")
        iota16 = jax.lax.broadcasted_iota(jnp.int32, (16,), 0)

        def start_gather(h, buf, sem):
            pltpu.async_copy(
                table_hbm.at[idx_v.at[h // 8, h % 8]], buf, sem
            )

        def wait_gather(buf, sem):
            pltpu.make_async_copy(table_hbm.at[idx_v.at[0, 0]], buf, sem).wait()

        def start_wout(h, ob, sem):
            for dq in range(_D // 8):
                pltpu.async_copy(
                    ob.at[pl.ds(dq * 8, 8)], out_hbm.at[h, dq, w], sem
                )

        def wait_wout(ob, sem):
            for dq in range(_D // 8):
                pltpu.make_async_copy(
                    ob.at[pl.ds(0, 8)], out_hbm.at[0, 0, w], sem
                ).wait()

        def transpose_scale(rows, ob):
            @plsc.parallel_loop(0, 32, 1)
            def _(i):
                r0 = (i // 4) * 16
                t0 = (i % 4) * 16
                rid = r0 + iota16
                for k in range(16):
                    cid = t0 + ((iota16 + k) & 15)
                    v = plsc.load_gather(rows, [rid, cid])
                    plsc.store_scatter(ob, [cid, rid], v * _SCALE)

        pltpu.sync_copy(idx_hbm.at[:, w], idx_v)
        start_gather(0, rows0, sg0)

        def body(j2, carry):
            h0 = 2 * j2
            @pl.when(j2 > 0)
            def _():
                wait_wout(ob1, sw1)
            start_gather(h0 + 1, rows1, sg1)
            wait_gather(rows0, sg0)
            @pl.when(j2 > 0)
            def _():
                wait_wout(ob0, sw0)
            transpose_scale(rows0, ob0)
            start_wout(h0, ob0, sw0)
            @pl.when(h0 + 2 < H)
            def _():
                start_gather(h0 + 2, rows0, sg0)
            wait_gather(rows1, sg1)
            transpose_scale(rows1, ob1)
            start_wout(h0 + 1, ob1, sw1)
            return carry

        lax.fori_loop(0, H // 2, body, 0)
        wait_wout(ob0, sw0)
        wait_wout(ob1, sw1)

    return emb_kernel


def kernel(x, table):
    B, H = x.shape
    BB, HQ = B // 128, H // 8
    phys_x = (
        x.astype(jnp.int32).reshape(BB, 128, HQ, 8).transpose(2, 0, 3, 1)
    )
    phys_out = _make_kernel(table.shape[0], BB, HQ)(table, phys_x)
    return phys_out.transpose(2, 4, 0, 1, 3).reshape(B, H, _D)


# speedup vs baseline: 1.2648x; 1.2648x over previous
import functools
import jax
import jax.numpy as jnp
from jax import lax
from jax.experimental import pallas as pl
from jax.experimental.pallas import tpu as pltpu
from jax.experimental.pallas import tpu_sc as plsc

_D = 64
_SCALE = 8.0

_info = plsc.get_sparse_core_info()
_NC, _NS = _info.num_cores, _info.num_subcores
_NW = _NC * _NS


def _make_kernel(V, BB, HQ):
    H = 8 * HQ
    mesh = plsc.VectorSubcoreMesh(core_axis_name="c", subcore_axis_name="s")

    @functools.partial(
        pl.kernel,
        mesh=mesh,
        out_type=jax.ShapeDtypeStruct((H, _D // 8, BB, 8, 128), jnp.float32),
        scratch_types=[
            pltpu.VMEM((HQ, 8, 128), jnp.int32),
            pltpu.VMEM((128, _D), jnp.float32),
            pltpu.VMEM((128, _D), jnp.float32),
            pltpu.VMEM((_D // 8, 8, 128), jnp.float32),
            pltpu.VMEM((_D // 8, 8, 128), jnp.float32),
            pltpu.SemaphoreType.DMA,
            pltpu.SemaphoreType.DMA,
            pltpu.SemaphoreType.DMA,
            pltpu.SemaphoreType.DMA,
        ],
        compiler_params=pltpu.CompilerParams(
            use_tc_tiling_on_sc=False, needs_layout_passes=False
        ),
    )
    def emb_kernel(
        table_hbm, idx_hbm, out_hbm,
        idx_v, rows0, rows1, ob0, ob1, sg0, sg1, sw0, sw1,
    ):
        w = lax.axis_index("s") * _NC + lax.axis_index("c")
        iota16 = jax.lax.broadcasted_iota(jnp.int32, (16,), 0)

        def start_gather(h, buf, sem):
            pltpu.async_copy(
                table_hbm.at[idx_v.at[h // 8, h % 8]], buf, sem
            )

        def wait_gather(buf, sem):
            pltpu.make_async_copy(table_hbm.at[idx_v.at[0, 0]], buf, sem).wait()

        def start_wout(h, ob, sem):
            pltpu.async_copy(ob, out_hbm.at[h, :, w], sem)

        def wait_wout(ob, sem):
            pltpu.make_async_copy(ob, out_hbm.at[0, :, w], sem).wait()

        def transpose_scale(rows, ob):
            @plsc.parallel_loop(0, 32, 1, unroll=2)
            def _(i):
                r0 = (i // 4) * 16
                t0 = (i % 4) * 16
                rid = r0 + iota16
                for k in range(16):
                    m = (iota16 + k) & 15
                    cid = t0 + m
                    v = plsc.load_gather(rows, [rid, cid])
                    plsc.store_scatter(ob, [cid >> 3, cid & 7, rid], v * _SCALE)

        pltpu.sync_copy(idx_hbm.at[:, w], idx_v)
        start_gather(0, rows0, sg0)

        def body(j2, carry):
            h0 = 2 * j2
            @pl.when(j2 > 0)
            def _():
                wait_wout(ob1, sw1)
            start_gather(h0 + 1, rows1, sg1)
            wait_gather(rows0, sg0)
            @pl.when(j2 > 0)
            def _():
                wait_wout(ob0, sw0)
            transpose_scale(rows0, ob0)
            start_wout(h0, ob0, sw0)
            @pl.when(h0 + 2 < H)
            def _():
                start_gather(h0 + 2, rows0, sg0)
            wait_gather(rows1, sg1)
            transpose_scale(rows1, ob1)
            start_wout(h0 + 1, ob1, sw1)
            return carry

        lax.fori_loop(0, H // 2, body, 0)
        wait_wout(ob0, sw0)
        wait_wout(ob1, sw1)

    return emb_kernel


def kernel(x, table):
    B, H = x.shape
    BB, HQ = B // 128, H // 8
    phys_x = (
        x.astype(jnp.int32).reshape(BB, 128, HQ, 8).transpose(2, 0, 3, 1)
    )
    phys_out = _make_kernel(table.shape[0], BB, HQ)(table, phys_x)
    return phys_out.transpose(2, 4, 0, 1, 3).reshape(B, H, _D)


# speedup vs baseline: 1.2963x; 1.0249x over previous
import functools
import jax
import jax.numpy as jnp
from jax import lax
from jax.experimental import pallas as pl
from jax.experimental.pallas import tpu as pltpu
from jax.experimental.pallas import tpu_sc as plsc

_D = 64
_SCALE = 8.0

_info = plsc.get_sparse_core_info()
_NC, _NS = _info.num_cores, _info.num_subcores
_NW = _NC * _NS


def _make_kernel(V, BB, HQ):
    H = 8 * HQ
    mesh = plsc.VectorSubcoreMesh(core_axis_name="c", subcore_axis_name="s")

    @functools.partial(
        pl.kernel,
        mesh=mesh,
        out_type=jax.ShapeDtypeStruct((H, _D // 8, BB, 8, 128), jnp.float32),
        scratch_types=[
            pltpu.VMEM((HQ, 8, 128), jnp.int32),
            pltpu.VMEM((128, _D), jnp.float32),
            pltpu.VMEM((128, _D), jnp.float32),
            pltpu.VMEM((_D // 8, 8, 128), jnp.float32),
            pltpu.VMEM((_D // 8, 8, 128), jnp.float32),
            pltpu.SemaphoreType.DMA,
            pltpu.SemaphoreType.DMA,
            pltpu.SemaphoreType.DMA,
            pltpu.SemaphoreType.DMA,
        ],
        compiler_params=pltpu.CompilerParams(
            use_tc_tiling_on_sc=False, needs_layout_passes=False
        ),
    )
    def emb_kernel(
        table_hbm, idx_hbm, out_hbm,
        idx_v, rows0, rows1, ob0, ob1, sg0, sg1, sw0, sw1,
    ):
        w = lax.axis_index("s") * _NC + lax.axis_index("c")
        iota16 = jax.lax.broadcasted_iota(jnp.int32, (16,), 0)

        def start_gather(h, buf, sem):
            pltpu.async_copy(
                table_hbm.at[idx_v.at[h // 8, h % 8]], buf, sem
            )

        def wait_gather(buf, sem):
            pltpu.make_async_copy(table_hbm.at[idx_v.at[0, 0]], buf, sem).wait()

        def start_wout(h, ob, sem):
            pltpu.async_copy(ob, out_hbm.at[h, :, w], sem)

        def wait_wout(ob, sem):
            pltpu.make_async_copy(ob, out_hbm.at[0, :, w], sem).wait()

        def transpose_scale(rows, ob):
            @plsc.parallel_loop(0, 32, 1)
            def _(i):
                r0 = (i // 4) * 16
                t0 = (i % 4) * 16
                rid = r0 + iota16
                for k in range(16):
                    m = (iota16 + k) & 15
                    cid = t0 + m
                    v = plsc.load_gather(rows, [rid, cid])
                    plsc.store_scatter(ob, [cid >> 3, cid & 7, rid], v * _SCALE)

        pltpu.sync_copy(idx_hbm.at[:, w], idx_v)
        start_gather(0, rows0, sg0)

        def body(j2, carry):
            h0 = 2 * j2
            @pl.when(j2 > 0)
            def _():
                wait_wout(ob1, sw1)
            start_gather(h0 + 1, rows1, sg1)
            wait_gather(rows0, sg0)
            @pl.when(j2 > 0)
            def _():
                wait_wout(ob0, sw0)
            transpose_scale(rows0, ob0)
            start_wout(h0, ob0, sw0)
            @pl.when(h0 + 2 < H)
            def _():
                start_gather(h0 + 2, rows0, sg0)
            wait_gather(rows1, sg1)
            transpose_scale(rows1, ob1)
            start_wout(h0 + 1, ob1, sw1)
            return carry

        lax.fori_loop(0, H // 2, body, 0)
        wait_wout(ob0, sw0)
        wait_wout(ob1, sw1)

    return emb_kernel


def kernel(x, table):
    B, H = x.shape
    BB, HQ = B // 128, H // 8
    phys_x = (
        x.astype(jnp.int32).reshape(BB, 128, HQ, 8).transpose(2, 0, 3, 1)
    )
    phys_out = _make_kernel(table.shape[0], BB, HQ)(table, phys_x)
    return phys_out.transpose(2, 4, 0, 1, 3).reshape(B, H, _D)
